# baseline (speedup 1.0000x reference)
import functools

import jax
import jax.numpy as jnp
from jax import lax
from jax.experimental import pallas as pl
from jax.experimental.pallas import tpu as pltpu
from jax.experimental.pallas import tpu_sc as plsc


def _emb_body(row_hbm, col_hbm, out_hbm, rbuf, cbuf, *, nc, rows_per_w, d, c):
    wid = lax.axis_index("s") * nc + lax.axis_index("c")
    base = wid * rows_per_w
    pltpu.sync_copy(row_hbm.at[pl.ds(pl.multiple_of(base * d, 8), rows_per_w * d)], rbuf)
    pltpu.sync_copy(col_hbm.at[pl.ds(pl.multiple_of(base * d, 8), rows_per_w * d)], cbuf)
    for r in range(rows_per_w):
        off = pl.multiple_of((base + r) * c, 8)
        pltpu.sync_copy(rbuf.at[pl.ds(r * d, d)], out_hbm.at[pl.ds(off, d)])
        pltpu.sync_copy(cbuf.at[pl.ds(r * d, d)],
                        out_hbm.at[pl.ds(pl.multiple_of((base + r) * c + d, 8), d)])


def _sc_emb(row_table, col_table):
    W, d = row_table.shape
    info = plsc.get_sparse_core_info()
    nc, ns = info.num_cores, info.num_subcores
    nw = nc * ns
    rows_per_w = W // nw
    mesh = plsc.VectorSubcoreMesh(core_axis_name="c", subcore_axis_name="s")
    k = functools.partial(
        pl.kernel,
        mesh=mesh,
        out_type=jax.ShapeDtypeStruct((W * 2 * d,), jnp.float32),
        scratch_types=[
            pltpu.VMEM((rows_per_w * d,), jnp.float32),
            pltpu.VMEM((rows_per_w * d,), jnp.float32),
        ],
    )(functools.partial(_emb_body, nc=nc, rows_per_w=rows_per_w, d=d, c=2 * d))
    return k(row_table.reshape(-1), col_table.reshape(-1)).reshape(W, 2 * d)


def _add_body(x_ref, emb_ref, o_ref):
    o_ref[...] = x_ref[...] + emb_ref[...][None, :, :]


def kernel(inputs, row_table, col_table):
    B, H, W, C = inputs.shape
    emb = _sc_emb(row_table, col_table)
    K = 32
    x = inputs.reshape(B * H, W, C)
    grid = (B * H // K,)
    out = pl.pallas_call(
        _add_body,
        grid=grid,
        in_specs=[
            pl.BlockSpec((K, W, C), lambda i: (i, 0, 0)),
            pl.BlockSpec((W, C), lambda i: (0, 0)),
        ],
        out_specs=pl.BlockSpec((K, W, C), lambda i: (i, 0, 0)),
        out_shape=jax.ShapeDtypeStruct((B * H, W, C), inputs.dtype),
    )(x, emb)
    return out.reshape(B, H, W, C)
